# Initial kernel scaffold; baseline (speedup 1.0000x reference)
import jax
import jax.numpy as jnp
from jax.experimental import pallas as pl


def kernel(input_ids, words, pos_table):
    raise NotImplementedError("write your pallas kernel here")


# speedup vs baseline: 1.1265x; 1.1265x over previous
import functools

import jax
import jax.numpy as jnp
from jax import lax
from jax.experimental import pallas as pl
from jax.experimental.pallas import tpu as pltpu
from jax.experimental.pallas import tpu_sc as plsc

B, S, D = 1024, 200, 64
N = B * S
NC, NS = 2, 16
NW = NC * NS
PER_W = N // NW
CH = 128
NCH = PER_W // CH
POS2 = 2 * S


def _emb_body(idx_hbm, pos2_hbm, words_hbm, out_hbm, idx_v, pos_v, buf, sem):
    cid = lax.axis_index("c")
    sid = lax.axis_index("s")
    wid = sid * NC + cid
    base = wid * PER_W
    pltpu.sync_copy(idx_hbm.at[pl.ds(base, PER_W)], idx_v)
    pltpu.sync_copy(pos2_hbm, pos_v)

    def chunk(c, carry):
        pltpu.async_copy(words_hbm.at[idx_v.at[pl.ds(c * CH, CH)]], buf, sem).wait()
        rbase = lax.rem(c * CH, S)

        def row(r, carry2):
            prow = rbase + r
            for j in range(4):
                sl = pl.ds(j * 16, 16)
                buf[r, sl] = buf[r, sl] + pos_v[prow, sl]
            return carry2

        lax.fori_loop(0, CH, row, None)
        pltpu.sync_copy(buf, out_hbm.at[pl.ds(base + c * CH, CH)])
        return carry

    lax.fori_loop(0, NCH, chunk, None)


def kernel(input_ids, words, pos_table):
    idx = input_ids.reshape(-1).astype(jnp.int32)
    pos2 = jnp.concatenate([pos_table[:S], pos_table[:S]], axis=0)
    mesh = plsc.VectorSubcoreMesh(core_axis_name="c", subcore_axis_name="s")
    out = pl.kernel(
        _emb_body,
        out_type=jax.ShapeDtypeStruct((N, D), jnp.float32),
        mesh=mesh,
        scratch_types=[
            pltpu.VMEM((PER_W,), jnp.int32),
            pltpu.VMEM((POS2, D), jnp.float32),
            pltpu.VMEM((CH, D), jnp.float32),
            pltpu.SemaphoreType.DMA,
        ],
        compiler_params=pltpu.CompilerParams(use_tc_tiling_on_sc=False),
    )(idx, pos2, words)
    return out.reshape(B, S, D)


# speedup vs baseline: 1.1934x; 1.0594x over previous
import functools

import jax
import jax.numpy as jnp
from jax import lax
from jax.experimental import pallas as pl
from jax.experimental.pallas import tpu as pltpu
from jax.experimental.pallas import tpu_sc as plsc

B, S, D = 1024, 200, 64
N = B * S
NC, NS = 2, 16
NW = NC * NS
PER_W = N // NW
CH = 128
NCH = PER_W // CH
POS2 = 2 * S
UNROLL = 8


def _emb_body(idx_hbm, pos2_hbm, words_hbm, out_hbm,
              idx_v, pos_v, g0, g1, o_v, sem0, sem1):
    cid = lax.axis_index("c")
    sid = lax.axis_index("s")
    wid = sid * NC + cid
    base = wid * PER_W
    pltpu.sync_copy(idx_hbm.at[pl.ds(base, PER_W)], idx_v)
    pltpu.sync_copy(pos2_hbm, pos_v)

    def start_gather(c, buf, sem):
        return pltpu.async_copy(
            words_hbm.at[idx_v.at[pl.ds(c * CH, CH)]], buf, sem)

    def wait_gather(buf, sem):
        pltpu.make_async_copy(words_hbm.at[idx_v.at[pl.ds(0, CH)]], buf,
                              sem).wait()

    def consume(c, buf, sem):
        wait_gather(buf, sem)
        rbase = lax.rem(c * CH, S)

        def add_row(r, carry):
            prow = rbase + r
            for j in range(4):
                sl = pl.ds(j * 16, 16)
                o_v[r, sl] = buf[r, sl] + pos_v[prow, sl]
            return carry

        lax.fori_loop(0, CH, add_row, None, unroll=UNROLL)

        @pl.when(c + 2 < NCH)
        def _():
            start_gather(c + 2, buf, sem)

        pltpu.sync_copy(o_v, out_hbm.at[pl.ds(base + c * CH, CH)])

    start_gather(0, g0, sem0)
    start_gather(1, g1, sem1)

    def step(t, carry):
        consume(2 * t, g0, sem0)
        consume(2 * t + 1, g1, sem1)
        return carry

    lax.fori_loop(0, NCH // 2, step, None)


def kernel(input_ids, words, pos_table):
    idx = input_ids.reshape(-1).astype(jnp.int32)
    pos2 = jnp.concatenate([pos_table[:S], pos_table[:S]], axis=0)
    mesh = plsc.VectorSubcoreMesh(core_axis_name="c", subcore_axis_name="s")
    out = pl.kernel(
        _emb_body,
        out_type=jax.ShapeDtypeStruct((N, D), jnp.float32),
        mesh=mesh,
        scratch_types=[
            pltpu.VMEM((PER_W,), jnp.int32),
            pltpu.VMEM((POS2, D), jnp.float32),
            pltpu.VMEM((CH, D), jnp.float32),
            pltpu.VMEM((CH, D), jnp.float32),
            pltpu.VMEM((CH, D), jnp.float32),
            pltpu.SemaphoreType.DMA,
            pltpu.SemaphoreType.DMA,
        ],
        compiler_params=pltpu.CompilerParams(use_tc_tiling_on_sc=False),
    )(idx, pos2, words)
    return out.reshape(B, S, D)
